# Initial kernel scaffold; baseline (speedup 1.0000x reference)
import jax
import jax.numpy as jnp
from jax.experimental import pallas as pl


def kernel(storage, vals, pointer):
    raise NotImplementedError("write your pallas kernel here")


# speedup vs baseline: 1.2365x; 1.2365x over previous
import functools

import jax
import jax.numpy as jnp
from jax import lax
from jax.experimental import pallas as pl
from jax.experimental.pallas import tpu as pltpu
from jax.experimental.pallas import tpu_sc as plsc
from jax._src.pallas import mpmd as _mpmd


@functools.lru_cache(maxsize=None)
def _make_scatter(capacity: int, n: int, dims: int):
  info = plsc.get_sparse_core_info()
  nc, ns, lanes = info.num_cores, info.num_subcores, info.num_lanes
  nw = nc * ns
  assert n % nw == 0, (n, nw)
  rows_per_w = n // nw
  assert rows_per_w % lanes == 0
  mesh = plsc.VectorSubcoreMesh(core_axis_name="c", subcore_axis_name="s")

  def body(storage_ref, vals_ref, ptr_ref, out_ref, idx_v, vals_v, ptr_v, sem):
    del storage_ref
    wid = lax.axis_index("s") * nc + lax.axis_index("c")
    base = wid * rows_per_w
    pltpu.sync_copy(vals_ref.at[pl.ds(base, rows_per_w)], vals_v)
    pltpu.sync_copy(ptr_ref, ptr_v)
    p = ptr_v[...]
    for i in range(rows_per_w // lanes):
      off = base + i * lanes
      idx_v[pl.ds(i * lanes, lanes)] = lax.rem(
          p + off + lax.iota(jnp.int32, lanes), capacity
      )
    pltpu.async_copy(vals_v, out_ref.at[idx_v], sem).wait()

  return _mpmd._mpmd_map(
      [(mesh, body)],
      out_types=jax.ShapeDtypeStruct((capacity, dims), jnp.float32),
      input_output_aliases={0: 0},
      scratch_types=[
          pltpu.VMEM((rows_per_w,), jnp.int32),
          pltpu.VMEM((rows_per_w, dims), jnp.float32),
          pltpu.VMEM((16,), jnp.int32),
          pltpu.SemaphoreType.DMA,
      ],
      name="fifo_scatter",
  )


def kernel(storage, vals, pointer):
  capacity, dims = storage.shape
  n = vals.shape[0]
  ptr_vec = jnp.broadcast_to(jnp.asarray(pointer, jnp.int32), (16,))
  scatter = _make_scatter(capacity, n, dims)
  return scatter(storage, vals, ptr_vec)
